# baseline (speedup 1.0000x reference)
import functools

import jax
import jax.numpy as jnp
from jax import lax
from jax.experimental import pallas as pl
from jax.experimental.pallas import tpu as pltpu
from jax.experimental.pallas import tpu_sc as plsc


_CHUNK = 128


def _argmin_body(x_ref, cb_ref, idx_ref):
    x = x_ref[...]
    cb = cb_ref[...]
    m, k = x.shape[0], cb.shape[0]
    x_sq = jnp.sum(x * x, axis=-1, keepdims=True)
    c_sq = jnp.sum(cb * cb, axis=-1)
    dots = lax.dot_general(x, cb, (((1,), (1,)), ((), ())),
                           preferred_element_type=jnp.float32)
    lane = lax.broadcasted_iota(jnp.int32, (m, _CHUNK), 1)
    best_v = None
    best_i = None
    for c in range(0, k, _CHUNK):
        dc = dots[:, c:c + _CHUNK]
        d2 = x_sq + c_sq[None, c:c + _CHUNK] - (dc + dc)
        l2 = jnp.sqrt(jnp.maximum(d2, 0.0))
        gidx = lane + c
        if best_v is None:
            best_v, best_i = l2, gidx
        else:
            lt = l2 < best_v
            best_v = jnp.where(lt, l2, best_v)
            best_i = jnp.where(lt, gidx, best_i)
    mn = jnp.min(best_v, axis=-1, keepdims=True)
    idx = jnp.min(jnp.where(best_v == mn, best_i, k), axis=-1)
    idx_ref[...] = idx.astype(jnp.int32)[None, None, :]


def _nearest_indices(x2d, codebook, tile_m, row_start, num_rows):
    d = x2d.shape[1]
    k = codebook.shape[0]
    grid = num_rows // tile_m
    blk0 = row_start // tile_m
    return pl.pallas_call(
        _argmin_body,
        grid=(grid,),
        in_specs=[
            pl.BlockSpec((tile_m, d), lambda i: (blk0 + i, 0)),
            pl.BlockSpec((k, d), lambda i: (0, 0)),
        ],
        out_specs=pl.BlockSpec((1, 1, tile_m), lambda i: (i, 0, 0)),
        out_shape=jax.ShapeDtypeStruct((grid, 1, tile_m), jnp.int32),
    )(x2d, codebook)


def _make_sc_gather(n, d):
    info = plsc.get_sparse_core_info()
    nw = info.num_cores * info.num_subcores
    b_per_w = n // nw
    mesh = plsc.VectorSubcoreMesh(core_axis_name="c", subcore_axis_name="s")

    @functools.partial(
        pl.kernel, mesh=mesh,
        out_type=jax.ShapeDtypeStruct((n, d), jnp.float32),
        compiler_params=pltpu.CompilerParams(use_tc_tiling_on_sc=False),
        scratch_types=[
            pltpu.VMEM((b_per_w,), jnp.int32),
            pltpu.VMEM((b_per_w, d), jnp.float32),
            pltpu.SemaphoreType.DMA,
        ],
    )
    def gather(table_hbm, idx_hbm, out_hbm, idx_v, rows_v, sem):
        wid = lax.axis_index("s") * info.num_cores + lax.axis_index("c")
        base = wid * b_per_w
        pltpu.sync_copy(idx_hbm.at[pl.ds(base, b_per_w)], idx_v)
        pltpu.async_copy(table_hbm.at[idx_v], rows_v, sem).wait()
        pltpu.sync_copy(rows_v, out_hbm.at[pl.ds(base, b_per_w)])

    return gather


def kernel(inputs, codebook):
    b, t, d = inputs.shape
    n = b * t
    half = n // 2
    x2d = inputs.reshape(n, d)
    gather = _make_sc_gather(half, d)
    idx0 = _nearest_indices(x2d, codebook, 2048, 0, half).reshape(half)
    q0 = gather(codebook, idx0)
    idx1 = _nearest_indices(x2d, codebook, 2048, half, half).reshape(half)
    q1 = gather(codebook, idx1)
    return jnp.concatenate([q0, q1], axis=0).reshape(b, t, d)


# speedup vs baseline: 1.7472x; 1.7472x over previous
import functools

import jax
import jax.numpy as jnp
from jax import lax
from jax.experimental import pallas as pl
from jax.experimental.pallas import tpu as pltpu
from jax.experimental.pallas import tpu_sc as plsc


_CHUNK = 128


def _argmin_body(x_ref, cb_ref, idx_ref):
    x = x_ref[...]
    cb = cb_ref[...]
    m, k = x.shape[0], cb.shape[0]
    x_sq = jnp.sum(x * x, axis=-1, keepdims=True)
    c_sq = jnp.sum(cb * cb, axis=-1)
    dots = lax.dot_general(x, cb, (((1,), (1,)), ((), ())),
                           preferred_element_type=jnp.float32)
    lane = lax.broadcasted_iota(jnp.int32, (m, _CHUNK), 1)
    best_v = None
    best_i = None
    for c in range(0, k, _CHUNK):
        dc = dots[:, c:c + _CHUNK]
        d2 = x_sq + c_sq[None, c:c + _CHUNK] - (dc + dc)
        l2 = jnp.sqrt(jnp.maximum(d2, 0.0))
        gidx = lane + c
        if best_v is None:
            best_v, best_i = l2, gidx
        else:
            lt = l2 < best_v
            best_v = jnp.where(lt, l2, best_v)
            best_i = jnp.where(lt, gidx, best_i)
    mn = jnp.min(best_v, axis=-1, keepdims=True)
    idx = jnp.min(jnp.where(best_v == mn, best_i, k), axis=-1)
    idx_ref[...] = idx.astype(jnp.int32)[None, None, :]


def _nearest_indices(x2d, codebook, tile_m, row_start, num_rows):
    d = x2d.shape[1]
    k = codebook.shape[0]
    grid = num_rows // tile_m
    blk0 = row_start // tile_m
    return pl.pallas_call(
        _argmin_body,
        grid=(grid,),
        in_specs=[
            pl.BlockSpec((tile_m, d), lambda i: (blk0 + i, 0)),
            pl.BlockSpec((k, d), lambda i: (0, 0)),
        ],
        out_specs=pl.BlockSpec((1, 1, tile_m), lambda i: (i, 0, 0)),
        out_shape=jax.ShapeDtypeStruct((grid, 1, tile_m), jnp.int32),
    )(x2d, codebook)


def _make_sc_gather(n, d):
    info = plsc.get_sparse_core_info()
    nw = info.num_cores * info.num_subcores
    b_per_w = n // nw
    mesh = plsc.VectorSubcoreMesh(core_axis_name="c", subcore_axis_name="s")

    @functools.partial(
        pl.kernel, mesh=mesh,
        out_type=jax.ShapeDtypeStruct((n, d), jnp.float32),
        compiler_params=pltpu.CompilerParams(use_tc_tiling_on_sc=False),
        scratch_types=[
            pltpu.VMEM((b_per_w,), jnp.int32),
            pltpu.VMEM((b_per_w, d), jnp.float32),
            pltpu.SemaphoreType.DMA,
        ],
    )
    def gather(table_hbm, idx_hbm, out_hbm, idx_v, rows_v, sem):
        wid = lax.axis_index("s") * info.num_cores + lax.axis_index("c")
        base = wid * b_per_w
        pltpu.sync_copy(idx_hbm.at[pl.ds(base, b_per_w)], idx_v)
        pltpu.async_copy(table_hbm.at[idx_v], rows_v, sem).wait()
        pltpu.sync_copy(rows_v, out_hbm.at[pl.ds(base, b_per_w)])

    return gather


def kernel(inputs, codebook):
    b, t, d = inputs.shape
    n = b * t
    half = n // 2
    x2d = inputs.reshape(n, d)
    gather = _make_sc_gather(half, d)
    idx = _nearest_indices(x2d, codebook, 2048, 0, n).reshape(n)
    return idx
